# baseline (device time: 97147 ns/iter reference)
import jax
import jax.numpy as jnp
from jax import lax
from jax.experimental import pallas as pl
from jax.experimental.pallas import tpu as pltpu

B = 8
H = 8
D = 128
BS = 16
NT = 512
NB_LOCAL = 512
PAGES_PER_CHUNK = 64
NC = NB_LOCAL // PAGES_PER_CHUNK
KEYS = PAGES_PER_CHUNK * BS
SCALE = D ** -0.5
NEG = -1e30


def kernel(Q, K, V, bt, lens):
    K3 = K.reshape(NB_LOCAL * BS, H, D)
    V3 = V.reshape(NB_LOCAL * BS, H, D)
    Q3 = Q.reshape(B, H, D)
    btT = bt.T
    lens2 = lens.reshape(1, B)

    def body(q_ref, k_ref, v_ref, btT_ref, lens_ref, out_ref,
             wchunk_scr, e64_scr, m_scr, l_scr, acc_scr,
             recv_m, recv_l, recv_acc, send_sems, recv_sems):
        c = pl.program_id(0)
        my_x = lax.axis_index("x")
        my_y = lax.axis_index("y")
        my_z = lax.axis_index("z")
        partner = (my_x, 1 - my_y, my_z)

        @pl.when(c == 0)
        def _init():
            barrier = pltpu.get_barrier_semaphore()
            pl.semaphore_signal(
                barrier, inc=1,
                device_id=partner, device_id_type=pl.DeviceIdType.MESH,
            )
            pl.semaphore_wait(barrier, 1)
            m_scr[...] = jnp.full((H * B, 1), NEG, jnp.float32)
            l_scr[...] = jnp.zeros((H * B, 1), jnp.float32)
            acc_scr[...] = jnp.zeros((H * B, D), jnp.float32)
            kpage = lax.broadcasted_iota(jnp.int32, (PAGES_PER_CHUNK, KEYS), 1) // BS
            prow = lax.broadcasted_iota(jnp.int32, (PAGES_PER_CHUNK, KEYS), 0)
            e64_scr[...] = (kpage == prow).astype(jnp.bfloat16)

        base = my_y * NB_LOCAL + c * PAGES_PER_CHUNK
        pid_row = base + lax.broadcasted_iota(
            jnp.int32, (1, PAGES_PER_CHUNK), 1)
        jidx = lax.broadcasted_iota(jnp.int32, (NT, 1), 0)
        for i in range(B):
            col = btT_ref[:, i:i + 1]
            hit = (col == pid_row) & (jidx < lens_ref[0, i])
            wchunk_scr[i:i + 1, :] = jnp.sum(
                hit.astype(jnp.float32), axis=0, keepdims=True)

        w_exp = lax.dot_general(
            wchunk_scr[...].astype(jnp.bfloat16), e64_scr[...],
            (((1,), (0,)), ((), ())), preferred_element_type=jnp.float32)
        wpos = w_exp > 0.0

        for h in range(H):
            q_h = q_ref[:, h, :].astype(jnp.bfloat16)
            k_h = k_ref[:, h, :].astype(jnp.bfloat16)
            v_h = v_ref[:, h, :].astype(jnp.bfloat16)
            s = lax.dot_general(
                q_h, k_h, (((1,), (1,)), ((), ())),
                preferred_element_type=jnp.float32) * SCALE
            s_m = jnp.where(wpos, s, NEG)
            r = slice(h * B, (h + 1) * B)
            m_old = m_scr[r, :]
            m_new = jnp.maximum(m_old, jnp.max(s_m, axis=1, keepdims=True))
            alpha = jnp.exp(m_old - m_new)
            p = w_exp * jnp.exp(s_m - m_new)
            l_new = l_scr[r, :] * alpha + jnp.sum(p, axis=1, keepdims=True)
            pv = lax.dot_general(
                p.astype(jnp.bfloat16), v_h, (((1,), (0,)), ((), ())),
                preferred_element_type=jnp.float32)
            acc_scr[r, :] = acc_scr[r, :] * alpha + pv
            m_scr[r, :] = m_new
            l_scr[r, :] = l_new

        @pl.when(c == NC - 1)
        def _exchange():
            rd_m = pltpu.make_async_remote_copy(
                src_ref=m_scr, dst_ref=recv_m,
                send_sem=send_sems.at[0], recv_sem=recv_sems.at[0],
                device_id=partner, device_id_type=pl.DeviceIdType.MESH)
            rd_l = pltpu.make_async_remote_copy(
                src_ref=l_scr, dst_ref=recv_l,
                send_sem=send_sems.at[1], recv_sem=recv_sems.at[1],
                device_id=partner, device_id_type=pl.DeviceIdType.MESH)
            rd_a = pltpu.make_async_remote_copy(
                src_ref=acc_scr, dst_ref=recv_acc,
                send_sem=send_sems.at[2], recv_sem=recv_sems.at[2],
                device_id=partner, device_id_type=pl.DeviceIdType.MESH)
            rd_m.start()
            rd_l.start()
            rd_a.start()
            rd_m.wait()
            rd_l.wait()
            rd_a.wait()

            m_s = m_scr[...]
            m_o = recv_m[...]
            m12 = jnp.maximum(m_s, m_o)
            c_s = jnp.exp(m_s - m12)
            c_o = jnp.exp(m_o - m12)
            l12 = l_scr[...] * c_s + recv_l[...] * c_o
            acc12 = acc_scr[...] * c_s + recv_acc[...] * c_o
            outv = acc12 / l12
            for h in range(H):
                out_ref[:, 0, h, :] = outv[h * B:(h + 1) * B, :]

    return pl.pallas_call(
        body,
        grid=(NC,),
        in_specs=[
            pl.BlockSpec((B, H, D), lambda c: (0, 0, 0)),
            pl.BlockSpec((KEYS, H, D), lambda c: (c, 0, 0)),
            pl.BlockSpec((KEYS, H, D), lambda c: (c, 0, 0)),
            pl.BlockSpec((NT, B), lambda c: (0, 0)),
            pl.BlockSpec(memory_space=pltpu.SMEM),
        ],
        out_specs=pl.BlockSpec((B, 1, H, D), lambda c: (0, 0, 0, 0)),
        out_shape=jax.ShapeDtypeStruct((B, 1, H, D), jnp.float32),
        scratch_shapes=[
            pltpu.VMEM((B, PAGES_PER_CHUNK), jnp.float32),
            pltpu.VMEM((PAGES_PER_CHUNK, KEYS), jnp.bfloat16),
            pltpu.VMEM((H * B, 1), jnp.float32),
            pltpu.VMEM((H * B, 1), jnp.float32),
            pltpu.VMEM((H * B, D), jnp.float32),
            pltpu.VMEM((H * B, 1), jnp.float32),
            pltpu.VMEM((H * B, 1), jnp.float32),
            pltpu.VMEM((H * B, D), jnp.float32),
            pltpu.SemaphoreType.DMA((3,)),
            pltpu.SemaphoreType.DMA((3,)),
        ],
        compiler_params=pltpu.CompilerParams(
            collective_id=0, dimension_semantics=("arbitrary",)),
    )(Q3, K3, V3, btT, lens2)


# device time: 81553 ns/iter; 1.1912x vs baseline; 1.1912x over previous
import jax
import jax.numpy as jnp
from jax import lax
from jax.experimental import pallas as pl
from jax.experimental.pallas import tpu as pltpu

B = 8
H = 8
D = 128
BS = 16
NT = 512
NB_LOCAL = 512
PAGES_PER_CHUNK = 64
NC = NB_LOCAL // PAGES_PER_CHUNK
KEYS = PAGES_PER_CHUNK * BS
HB = H * B
HD = H * D
SCALE = D ** -0.5
NEG = -1e30


def kernel(Q, K, V, bt, lens):
    K2 = K.reshape(NB_LOCAL * BS, HD)
    V2 = V.reshape(NB_LOCAL * BS, HD)
    Q3 = Q.reshape(B, H, D)
    btT = bt.T
    lens2 = lens.reshape(1, B)

    def body(q_ref, k_ref, v_ref, btT_ref, lens_ref, out_ref,
             qblk_scr, wchunk_scr, e64_scr, t64_scr,
             m_scr, l_scr, acc_scr, accd_scr,
             recv_m, recv_l, recv_acc, send_sems, recv_sems):
        c = pl.program_id(0)
        my_x = lax.axis_index("x")
        my_y = lax.axis_index("y")
        my_z = lax.axis_index("z")
        partner = (my_x, 1 - my_y, my_z)

        @pl.when(c == 0)
        def _init():
            barrier = pltpu.get_barrier_semaphore()
            pl.semaphore_signal(
                barrier, inc=1,
                device_id=partner, device_id_type=pl.DeviceIdType.MESH,
            )
            pl.semaphore_wait(barrier, 1)
            m_scr[...] = jnp.full((HB, 1), NEG, jnp.float32)
            l_scr[...] = jnp.zeros((HB, 1), jnp.float32)
            acc_scr[...] = jnp.zeros((HB, HD), jnp.float32)
            qblk_scr[...] = jnp.zeros((HB, HD), jnp.bfloat16)
            for h in range(H):
                qblk_scr[h * B:(h + 1) * B, h * D:(h + 1) * D] = (
                    q_ref[:, h, :].astype(jnp.bfloat16))
            kpage = lax.broadcasted_iota(jnp.int32, (PAGES_PER_CHUNK, KEYS), 1) // BS
            prow = lax.broadcasted_iota(jnp.int32, (PAGES_PER_CHUNK, KEYS), 0)
            e64_scr[...] = (kpage == prow).astype(jnp.bfloat16)
            rrow = lax.broadcasted_iota(jnp.int32, (HB, B), 0)
            bcol = lax.broadcasted_iota(jnp.int32, (HB, B), 1)
            t64_scr[...] = (rrow % B == bcol).astype(jnp.bfloat16)

        base = my_y * NB_LOCAL + c * PAGES_PER_CHUNK
        pid_row = base + lax.broadcasted_iota(
            jnp.int32, (1, PAGES_PER_CHUNK), 1)
        jidx = lax.broadcasted_iota(jnp.int32, (NT, 1), 0)
        for i in range(B):
            col = btT_ref[:, i:i + 1]
            hit = (col == pid_row) & (jidx < lens_ref[0, i])
            wchunk_scr[i:i + 1, :] = jnp.sum(
                hit.astype(jnp.float32), axis=0, keepdims=True)

        w_exp = lax.dot_general(
            wchunk_scr[...].astype(jnp.bfloat16), e64_scr[...],
            (((1,), (0,)), ((), ())), preferred_element_type=jnp.float32)
        w64 = lax.dot_general(
            t64_scr[...], w_exp.astype(jnp.bfloat16),
            (((1,), (0,)), ((), ())), preferred_element_type=jnp.float32)

        k_c = k_ref[...].astype(jnp.bfloat16)
        s = lax.dot_general(
            qblk_scr[...], k_c, (((1,), (1,)), ((), ())),
            preferred_element_type=jnp.float32) * SCALE
        s_m = jnp.where(w64 > 0.0, s, NEG)
        m_old = m_scr[...]
        m_new = jnp.maximum(m_old, jnp.max(s_m, axis=1, keepdims=True))
        alpha = jnp.exp(m_old - m_new)
        p = w64 * jnp.exp(s_m - m_new)
        l_scr[...] = l_scr[...] * alpha + jnp.sum(p, axis=1, keepdims=True)
        m_scr[...] = m_new
        pv = lax.dot_general(
            p.astype(jnp.bfloat16), v_ref[...].astype(jnp.bfloat16),
            (((1,), (0,)), ((), ())),
            preferred_element_type=jnp.float32)
        acc_scr[...] = acc_scr[...] * alpha + pv

        @pl.when(c == NC - 1)
        def _exchange():
            for h in range(H):
                r = slice(h * B, (h + 1) * B)
                accd_scr[r, :] = acc_scr[r, h * D:(h + 1) * D]

            rd_m = pltpu.make_async_remote_copy(
                src_ref=m_scr, dst_ref=recv_m,
                send_sem=send_sems.at[0], recv_sem=recv_sems.at[0],
                device_id=partner, device_id_type=pl.DeviceIdType.MESH)
            rd_l = pltpu.make_async_remote_copy(
                src_ref=l_scr, dst_ref=recv_l,
                send_sem=send_sems.at[1], recv_sem=recv_sems.at[1],
                device_id=partner, device_id_type=pl.DeviceIdType.MESH)
            rd_a = pltpu.make_async_remote_copy(
                src_ref=accd_scr, dst_ref=recv_acc,
                send_sem=send_sems.at[2], recv_sem=recv_sems.at[2],
                device_id=partner, device_id_type=pl.DeviceIdType.MESH)
            rd_m.start()
            rd_l.start()
            rd_a.start()
            rd_m.wait()
            rd_l.wait()
            rd_a.wait()

            m_s = m_scr[...]
            m_o = recv_m[...]
            m12 = jnp.maximum(m_s, m_o)
            c_s = jnp.exp(m_s - m12)
            c_o = jnp.exp(m_o - m12)
            l12 = l_scr[...] * c_s + recv_l[...] * c_o
            acc12 = accd_scr[...] * c_s + recv_acc[...] * c_o
            outv = acc12 / l12
            for h in range(H):
                out_ref[:, 0, h, :] = outv[h * B:(h + 1) * B, :]

    return pl.pallas_call(
        body,
        grid=(NC,),
        in_specs=[
            pl.BlockSpec((B, H, D), lambda c: (0, 0, 0)),
            pl.BlockSpec((KEYS, HD), lambda c: (c, 0)),
            pl.BlockSpec((KEYS, HD), lambda c: (c, 0)),
            pl.BlockSpec((NT, B), lambda c: (0, 0)),
            pl.BlockSpec(memory_space=pltpu.SMEM),
        ],
        out_specs=pl.BlockSpec((B, 1, H, D), lambda c: (0, 0, 0, 0)),
        out_shape=jax.ShapeDtypeStruct((B, 1, H, D), jnp.float32),
        scratch_shapes=[
            pltpu.VMEM((HB, HD), jnp.bfloat16),
            pltpu.VMEM((B, PAGES_PER_CHUNK), jnp.float32),
            pltpu.VMEM((PAGES_PER_CHUNK, KEYS), jnp.bfloat16),
            pltpu.VMEM((HB, B), jnp.bfloat16),
            pltpu.VMEM((HB, 1), jnp.float32),
            pltpu.VMEM((HB, 1), jnp.float32),
            pltpu.VMEM((HB, HD), jnp.float32),
            pltpu.VMEM((HB, D), jnp.float32),
            pltpu.VMEM((HB, 1), jnp.float32),
            pltpu.VMEM((HB, 1), jnp.float32),
            pltpu.VMEM((HB, D), jnp.float32),
            pltpu.SemaphoreType.DMA((3,)),
            pltpu.SemaphoreType.DMA((3,)),
        ],
        compiler_params=pltpu.CompilerParams(
            collective_id=0, dimension_semantics=("arbitrary",)),
    )(Q3, K2, V2, btT, lens2)


# device time: 37182 ns/iter; 2.6127x vs baseline; 2.1933x over previous
import jax
import jax.numpy as jnp
from jax import lax
from jax.experimental import pallas as pl
from jax.experimental.pallas import tpu as pltpu

B = 8
H = 8
D = 128
BS = 16
NT = 512
NB_LOCAL = 512
PAGES_PER_CHUNK = 64
NC = NB_LOCAL // PAGES_PER_CHUNK
KEYS = PAGES_PER_CHUNK * BS
HB = H * B
HD = H * D
SCALE = D ** -0.5
NEG = -1e30


def kernel(Q, K, V, bt, lens):
    K2 = K.reshape(NB_LOCAL * BS, H, D)
    V2 = V.reshape(NB_LOCAL * BS, H, D)
    Q3 = Q.reshape(B, H, D)
    btT = bt.T
    lens2 = lens.reshape(1, B)

    def body(q_ref, k_ref, v_ref, btT_ref, lens_ref, out_ref,
             qblk_scr, wchunk_scr, e64_scr, t64_scr,
             m_scr, l_scr, acc_scr, accd_scr,
             recv_m, recv_l, recv_acc, send_sems, recv_sems):
        c = pl.program_id(0)
        my_x = lax.axis_index("x")
        my_y = lax.axis_index("y")
        my_z = lax.axis_index("z")
        partner = (my_x, 1 - my_y, my_z)

        @pl.when(c == 0)
        def _init():
            barrier = pltpu.get_barrier_semaphore()
            pl.semaphore_signal(
                barrier, inc=1,
                device_id=partner, device_id_type=pl.DeviceIdType.MESH,
            )
            pl.semaphore_wait(barrier, 1)
            m_scr[...] = jnp.full((HB, 1), NEG, jnp.float32)
            l_scr[...] = jnp.zeros((HB, 1), jnp.float32)
            acc_scr[...] = jnp.zeros((HB, HD), jnp.float32)
            qblk_scr[...] = jnp.zeros((HB, HD), jnp.bfloat16)
            for h in range(H):
                qblk_scr[h * B:(h + 1) * B, h * D:(h + 1) * D] = (
                    q_ref[:, h, :].astype(jnp.bfloat16))
            kpage = lax.broadcasted_iota(jnp.int32, (PAGES_PER_CHUNK, KEYS), 1) // BS
            prow = lax.broadcasted_iota(jnp.int32, (PAGES_PER_CHUNK, KEYS), 0)
            e64_scr[...] = (kpage == prow).astype(jnp.bfloat16)
            rrow = lax.broadcasted_iota(jnp.int32, (HB, B), 0)
            bcol = lax.broadcasted_iota(jnp.int32, (HB, B), 1)
            t64_scr[...] = (rrow % B == bcol).astype(jnp.bfloat16)

        base = my_y * NB_LOCAL + c * PAGES_PER_CHUNK
        pid_row = base + lax.broadcasted_iota(
            jnp.int32, (1, PAGES_PER_CHUNK), 1)
        jidx = lax.broadcasted_iota(jnp.int32, (NT, 1), 0)
        for i in range(B):
            col = btT_ref[:, i:i + 1]
            hit = (col == pid_row) & (jidx < lens_ref[0, i])
            wchunk_scr[i:i + 1, :] = jnp.sum(
                hit.astype(jnp.float32), axis=0, keepdims=True)

        w_exp = lax.dot_general(
            wchunk_scr[...].astype(jnp.bfloat16), e64_scr[...],
            (((1,), (0,)), ((), ())), preferred_element_type=jnp.float32)
        w64 = lax.dot_general(
            t64_scr[...], w_exp.astype(jnp.bfloat16),
            (((1,), (0,)), ((), ())), preferred_element_type=jnp.float32)

        k_c = k_ref[...].reshape(KEYS, HD).astype(jnp.bfloat16)
        s = lax.dot_general(
            qblk_scr[...], k_c, (((1,), (1,)), ((), ())),
            preferred_element_type=jnp.float32) * SCALE
        s_m = jnp.where(w64 > 0.0, s, NEG)
        m_old = m_scr[...]
        m_new = jnp.maximum(m_old, jnp.max(s_m, axis=1, keepdims=True))
        alpha = jnp.exp(m_old - m_new)
        p = w64 * jnp.exp(s_m - m_new)
        l_scr[...] = l_scr[...] * alpha + jnp.sum(p, axis=1, keepdims=True)
        m_scr[...] = m_new
        pv = lax.dot_general(
            p.astype(jnp.bfloat16),
            v_ref[...].reshape(KEYS, HD).astype(jnp.bfloat16),
            (((1,), (0,)), ((), ())),
            preferred_element_type=jnp.float32)
        acc_scr[...] = acc_scr[...] * alpha + pv

        @pl.when(c == NC - 1)
        def _exchange():
            for h in range(H):
                r = slice(h * B, (h + 1) * B)
                accd_scr[r, :] = acc_scr[r, h * D:(h + 1) * D]

            rd_m = pltpu.make_async_remote_copy(
                src_ref=m_scr, dst_ref=recv_m,
                send_sem=send_sems.at[0], recv_sem=recv_sems.at[0],
                device_id=partner, device_id_type=pl.DeviceIdType.MESH)
            rd_l = pltpu.make_async_remote_copy(
                src_ref=l_scr, dst_ref=recv_l,
                send_sem=send_sems.at[1], recv_sem=recv_sems.at[1],
                device_id=partner, device_id_type=pl.DeviceIdType.MESH)
            rd_a = pltpu.make_async_remote_copy(
                src_ref=accd_scr, dst_ref=recv_acc,
                send_sem=send_sems.at[2], recv_sem=recv_sems.at[2],
                device_id=partner, device_id_type=pl.DeviceIdType.MESH)
            rd_m.start()
            rd_l.start()
            rd_a.start()
            rd_m.wait()
            rd_l.wait()
            rd_a.wait()

            m_s = m_scr[...]
            m_o = recv_m[...]
            m12 = jnp.maximum(m_s, m_o)
            c_s = jnp.exp(m_s - m12)
            c_o = jnp.exp(m_o - m12)
            l12 = l_scr[...] * c_s + recv_l[...] * c_o
            acc12 = accd_scr[...] * c_s + recv_acc[...] * c_o
            outv = acc12 / l12
            for h in range(H):
                out_ref[:, 0, h, :] = outv[h * B:(h + 1) * B, :]

    return pl.pallas_call(
        body,
        grid=(NC,),
        in_specs=[
            pl.BlockSpec((B, H, D), lambda c: (0, 0, 0)),
            pl.BlockSpec((KEYS, H, D), lambda c: (c, 0, 0)),
            pl.BlockSpec((KEYS, H, D), lambda c: (c, 0, 0)),
            pl.BlockSpec((NT, B), lambda c: (0, 0)),
            pl.BlockSpec(memory_space=pltpu.SMEM),
        ],
        out_specs=pl.BlockSpec((B, 1, H, D), lambda c: (0, 0, 0, 0)),
        out_shape=jax.ShapeDtypeStruct((B, 1, H, D), jnp.float32),
        scratch_shapes=[
            pltpu.VMEM((HB, HD), jnp.bfloat16),
            pltpu.VMEM((B, PAGES_PER_CHUNK), jnp.float32),
            pltpu.VMEM((PAGES_PER_CHUNK, KEYS), jnp.bfloat16),
            pltpu.VMEM((HB, B), jnp.bfloat16),
            pltpu.VMEM((HB, 1), jnp.float32),
            pltpu.VMEM((HB, 1), jnp.float32),
            pltpu.VMEM((HB, HD), jnp.float32),
            pltpu.VMEM((HB, D), jnp.float32),
            pltpu.VMEM((HB, 1), jnp.float32),
            pltpu.VMEM((HB, 1), jnp.float32),
            pltpu.VMEM((HB, D), jnp.float32),
            pltpu.SemaphoreType.DMA((3,)),
            pltpu.SemaphoreType.DMA((3,)),
        ],
        compiler_params=pltpu.CompilerParams(
            collective_id=0, dimension_semantics=("arbitrary",)),
    )(Q3, K2, V2, btT, lens2)
